# baseline (speedup 1.0000x reference)
import functools

import jax
import jax.numpy as jnp
import numpy as np
from jax import lax
from jax.experimental import pallas as pl
from jax.experimental.pallas import tpu as pltpu
from jax.experimental.pallas import tpu_sc as plsc

B, H, W, C = 4, 384, 384, 96
HP, WP = H // 2, W // 2
NWORK = 32
PAIRS_PER_W = (B * HP) // NWORK
TCOLS = W // 128
NCHUNK = PAIRS_PER_W * TCOLS

R2C = np.float32(1.0) / np.float32(2.0)
R3C = np.float32(1.0) / np.float32(3.0)
R4C = np.float32(1.0) / np.float32(4.0)
R5C = np.float32(1.0) / np.float32(5.0)


def _block_net(a, b, c, d, p):
    one = jnp.full((16,), 1.0, jnp.float32)
    zero = jnp.full((16,), 0.0, jnp.float32)
    two = jnp.full((16,), 2.0, jnp.float32)
    three = jnp.full((16,), 3.0, jnp.float32)
    xab = jnp.where(a >= b, one, zero)
    xac = jnp.where(a >= c, one, zero)
    xad = jnp.where(a >= d, one, zero)
    xbc = jnp.where(b >= c, one, zero)
    xbd = jnp.where(b >= d, one, zero)
    xcd = jnp.where(c >= d, one, zero)
    ra = three - (xab + xac + xad)
    rb = xab + (one - xbc) + (one - xbd)
    rc = xac + xbc + (one - xcd)
    rd = xad + xbd + xcd
    hi1 = jnp.maximum(a, b)
    lo1 = jnp.minimum(a, b)
    hi2 = jnp.maximum(c, d)
    lo2 = jnp.minimum(c, d)
    s0 = jnp.maximum(hi1, hi2)
    s3 = jnp.minimum(lo1, lo2)
    mhi = jnp.minimum(hi1, hi2)
    mlo = jnp.maximum(lo1, lo2)
    s1 = jnp.maximum(mhi, mlo)
    s2 = jnp.minimum(mhi, mlo)
    c1 = s0 + s1
    c2 = c1 + s2
    c3 = c2 + s3
    t0 = (s0 + p) * R2C
    t1 = (c1 + p) * R3C
    t2 = (c2 + p) * R4C
    t3 = (c3 + p) * R5C
    repl = jnp.maximum(jnp.maximum(t0, t1), jnp.maximum(t2, t3))
    am = jnp.where(t0 >= repl, zero,
                   jnp.where(t1 >= repl, one,
                             jnp.where(t2 >= repl, two, three)))
    oa = jnp.where(ra <= am, repl, a)
    ob = jnp.where(rb <= am, repl, b)
    oc = jnp.where(rc <= am, repl, c)
    od = jnp.where(rd <= am, repl, d)
    return oa, ob, oc, od, repl


@functools.partial(
    pl.kernel,
    out_type=(
        jax.ShapeDtypeStruct((B, H, C, W), jnp.float32),
        jax.ShapeDtypeStruct((B, HP, WP, C), jnp.float32),
    ),
    mesh=plsc.VectorSubcoreMesh(core_axis_name="c", subcore_axis_name="s"),
    compiler_params=pltpu.CompilerParams(use_tc_tiling_on_sc=True,
                                         needs_layout_passes=False),
    scratch_types=(
        [pltpu.VMEM((C, 128), jnp.float32) for _ in range(3)]
        + [pltpu.VMEM((C, 128), jnp.float32) for _ in range(3)]
        + [pltpu.VMEM((64, C), jnp.float32) for _ in range(3)]
        + [pltpu.SemaphoreType.DMA for _ in range(6)]
    ),
)
def _unpool_sc(x_hbm, p_hbm, ox_hbm, op_hbm,
               r0_0, r0_1, r0_2, r1_0, r1_1, r1_2,
               pp_0, pp_1, pp_2,
               is0, is1, is2, os0, os1, os2):
    bufs = ((r0_0, r1_0, pp_0), (r0_1, r1_1, pp_1), (r0_2, r1_2, pp_2))
    isems = (is0, is1, is2)
    osems = (os0, os1, os2)

    wid = lax.axis_index("s") * 2 + lax.axis_index("c")
    pair0 = wid * PAIRS_PER_W
    bb = wid // 8

    iota = lax.broadcasted_iota(jnp.int32, (16,), 0)

    def coords(t):
        rp = pair0 + t // TCOLS
        tc = t - (t // TCOLS) * TCOLS
        ii = rp - bb * HP
        return ii, tc

    def start_in(t, s):
        ii, tc = coords(t)
        r0, r1, pp = bufs[s]
        pltpu.async_copy(x_hbm.at[bb, 2 * ii, :, pl.ds(tc * 128, 128)],
                         r0, isems[s])
        pltpu.async_copy(x_hbm.at[bb, 2 * ii + 1, :, pl.ds(tc * 128, 128)],
                         r1, isems[s])
        pltpu.async_copy(p_hbm.at[bb, ii, pl.ds(tc * 64, 64), :],
                         pp, isems[s])

    def wait_in(s):
        r0, r1, pp = bufs[s]
        pltpu.make_async_copy(x_hbm.at[0, 0, :, pl.ds(0, 128)], r0,
                              isems[s]).wait()
        pltpu.make_async_copy(x_hbm.at[0, 0, :, pl.ds(0, 128)], r1,
                              isems[s]).wait()
        pltpu.make_async_copy(p_hbm.at[0, 0, pl.ds(0, 64), :], pp,
                              isems[s]).wait()

    def start_out(t, s):
        ii, tc = coords(t)
        r0, r1, pp = bufs[s]
        pltpu.async_copy(r0, ox_hbm.at[bb, 2 * ii, :, pl.ds(tc * 128, 128)],
                         osems[s])
        pltpu.async_copy(r1, ox_hbm.at[bb, 2 * ii + 1, :, pl.ds(tc * 128, 128)],
                         osems[s])
        pltpu.async_copy(pp, op_hbm.at[bb, ii, pl.ds(tc * 64, 64), :],
                         osems[s])

    def wait_out(s):
        r0, r1, pp = bufs[s]
        pltpu.make_async_copy(r0, ox_hbm.at[0, 0, :, pl.ds(0, 128)],
                              osems[s]).wait()
        pltpu.make_async_copy(r1, ox_hbm.at[0, 0, :, pl.ds(0, 128)],
                              osems[s]).wait()
        pltpu.make_async_copy(pp, op_hbm.at[0, 0, pl.ds(0, 64), :],
                              osems[s]).wait()

    def compute(s):
        r0, r1, pp = bufs[s]

        @plsc.parallel_loop(0, 4 * 16)
        def grp_body(i):
            j0 = lax.shift_left(lax.shift_right_logical(i, 4), 4)
            o = lax.bitwise_and(i, 15)
            jv = j0 + lax.bitwise_and(iota + o, 15)
            wav = jv * 2
            wbv = wav + 1
            for G in range(6):
                ccv = jnp.int32(G * 16) + iota
                a = plsc.load_gather(r0, [ccv, wav])
                b = plsc.load_gather(r0, [ccv, wbv])
                c = plsc.load_gather(r1, [ccv, wav])
                d = plsc.load_gather(r1, [ccv, wbv])
                p = plsc.load_gather(pp, [jv, ccv])
                oa, ob, oc, od, repl = _block_net(a, b, c, d, p)
                plsc.store_scatter(r0, [ccv, wav], oa)
                plsc.store_scatter(r0, [ccv, wbv], ob)
                plsc.store_scatter(r1, [ccv, wav], oc)
                plsc.store_scatter(r1, [ccv, wbv], od)
                plsc.store_scatter(pp, [jv, ccv], repl)

    start_in(jnp.int32(0), 0)
    start_in(jnp.int32(1), 1)

    def outer(u, carry):
        for s in range(3):
            t = u * 3 + s
            wait_in(s)
            compute(s)
            start_out(t, s)
            s2 = (s + 2) % 3

            @pl.when(t >= 1)
            def _():
                wait_out(s2)

            @pl.when(t + 2 < NCHUNK)
            def _():
                start_in(t + 2, s2)
        return carry

    lax.fori_loop(0, NCHUNK // 3, outer, 0)
    wait_out((NCHUNK - 1) % 3)


def kernel(x, pooled):
    xt = x.transpose(0, 1, 3, 2)
    oxt, op2 = _unpool_sc(xt, pooled)
    return oxt.transpose(0, 1, 3, 2), op2


# speedup vs baseline: 1.0499x; 1.0499x over previous
import functools

import jax
import jax.numpy as jnp
import numpy as np
from jax import lax
from jax.experimental import pallas as pl
from jax.experimental.pallas import tpu as pltpu
from jax.experimental.pallas import tpu_sc as plsc

B, H, W, C = 4, 384, 384, 96
HP, WP = H // 2, W // 2
NWORK = 32
PAIRS_PER_W = (B * HP) // NWORK
TCOLS = W // 128
NCHUNK = PAIRS_PER_W * TCOLS

R2C = np.float32(1.0) / np.float32(2.0)
R3C = np.float32(1.0) / np.float32(3.0)
R4C = np.float32(1.0) / np.float32(4.0)
R5C = np.float32(1.0) / np.float32(5.0)


def _block_net(a, b, c, d, p):
    one = jnp.full((16,), 1.0, jnp.float32)
    zero = jnp.full((16,), 0.0, jnp.float32)
    two = jnp.full((16,), 2.0, jnp.float32)
    three = jnp.full((16,), 3.0, jnp.float32)
    xab = jnp.where(a >= b, one, zero)
    xac = jnp.where(a >= c, one, zero)
    xad = jnp.where(a >= d, one, zero)
    xbc = jnp.where(b >= c, one, zero)
    xbd = jnp.where(b >= d, one, zero)
    xcd = jnp.where(c >= d, one, zero)
    ra = three - (xab + xac + xad)
    rb = xab + (one - xbc) + (one - xbd)
    rc = xac + xbc + (one - xcd)
    rd = xad + xbd + xcd
    hi1 = jnp.maximum(a, b)
    lo1 = jnp.minimum(a, b)
    hi2 = jnp.maximum(c, d)
    lo2 = jnp.minimum(c, d)
    s0 = jnp.maximum(hi1, hi2)
    s3 = jnp.minimum(lo1, lo2)
    mhi = jnp.minimum(hi1, hi2)
    mlo = jnp.maximum(lo1, lo2)
    s1 = jnp.maximum(mhi, mlo)
    s2 = jnp.minimum(mhi, mlo)
    c1 = s0 + s1
    c2 = c1 + s2
    c3 = c2 + s3
    t0 = (s0 + p) * R2C
    t1 = (c1 + p) * R3C
    t2 = (c2 + p) * R4C
    t3 = (c3 + p) * R5C
    repl = jnp.maximum(jnp.maximum(t0, t1), jnp.maximum(t2, t3))
    am = jnp.where(t0 >= repl, zero,
                   jnp.where(t1 >= repl, one,
                             jnp.where(t2 >= repl, two, three)))
    oa = jnp.where(ra <= am, repl, a)
    ob = jnp.where(rb <= am, repl, b)
    oc = jnp.where(rc <= am, repl, c)
    od = jnp.where(rd <= am, repl, d)
    return oa, ob, oc, od, repl


@functools.partial(
    pl.kernel,
    out_type=(
        jax.ShapeDtypeStruct((B, H, C, W), jnp.float32),
        jax.ShapeDtypeStruct((B, HP, WP, C), jnp.float32),
    ),
    mesh=plsc.VectorSubcoreMesh(core_axis_name="c", subcore_axis_name="s"),
    compiler_params=pltpu.CompilerParams(use_tc_tiling_on_sc=True,
                                         needs_layout_passes=False),
    scratch_types=(
        [pltpu.VMEM((C, 128), jnp.float32) for _ in range(3)]
        + [pltpu.VMEM((C, 128), jnp.float32) for _ in range(3)]
        + [pltpu.VMEM((64, C), jnp.float32) for _ in range(3)]
        + [pltpu.SemaphoreType.DMA for _ in range(6)]
    ),
)
def _unpool_sc(x_hbm, p_hbm, ox_hbm, op_hbm,
               r0_0, r0_1, r0_2, r1_0, r1_1, r1_2,
               pp_0, pp_1, pp_2,
               is0, is1, is2, os0, os1, os2):
    bufs = ((r0_0, r1_0, pp_0), (r0_1, r1_1, pp_1), (r0_2, r1_2, pp_2))
    isems = (is0, is1, is2)
    osems = (os0, os1, os2)

    wid = lax.axis_index("s") * 2 + lax.axis_index("c")
    pair0 = wid * PAIRS_PER_W
    bb = wid // 8

    iota = lax.broadcasted_iota(jnp.int32, (16,), 0)

    def coords(t):
        rp = pair0 + t // TCOLS
        tc = t - (t // TCOLS) * TCOLS
        ii = rp - bb * HP
        return ii, tc

    def start_in(t, s):
        ii, tc = coords(t)
        r0, r1, pp = bufs[s]
        pltpu.async_copy(x_hbm.at[bb, 2 * ii, :, pl.ds(tc * 128, 128)],
                         r0, isems[s])
        pltpu.async_copy(x_hbm.at[bb, 2 * ii + 1, :, pl.ds(tc * 128, 128)],
                         r1, isems[s])
        pltpu.async_copy(p_hbm.at[bb, ii, pl.ds(tc * 64, 64), :],
                         pp, isems[s])

    def wait_in(s):
        r0, r1, pp = bufs[s]
        pltpu.make_async_copy(x_hbm.at[0, 0, :, pl.ds(0, 128)], r0,
                              isems[s]).wait()
        pltpu.make_async_copy(x_hbm.at[0, 0, :, pl.ds(0, 128)], r1,
                              isems[s]).wait()
        pltpu.make_async_copy(p_hbm.at[0, 0, pl.ds(0, 64), :], pp,
                              isems[s]).wait()

    def start_out(t, s):
        ii, tc = coords(t)
        r0, r1, pp = bufs[s]
        pltpu.async_copy(r0, ox_hbm.at[bb, 2 * ii, :, pl.ds(tc * 128, 128)],
                         osems[s])
        pltpu.async_copy(r1, ox_hbm.at[bb, 2 * ii + 1, :, pl.ds(tc * 128, 128)],
                         osems[s])
        pltpu.async_copy(pp, op_hbm.at[bb, ii, pl.ds(tc * 64, 64), :],
                         osems[s])

    def wait_out(s):
        r0, r1, pp = bufs[s]
        pltpu.make_async_copy(r0, ox_hbm.at[0, 0, :, pl.ds(0, 128)],
                              osems[s]).wait()
        pltpu.make_async_copy(r1, ox_hbm.at[0, 0, :, pl.ds(0, 128)],
                              osems[s]).wait()
        pltpu.make_async_copy(pp, op_hbm.at[0, 0, pl.ds(0, 64), :],
                              osems[s]).wait()

    def compute(s):
        r0, r1, pp = bufs[s]

        @plsc.parallel_loop(0, 6 * 4 * 16)
        def grp_body(i):
            cc0 = lax.shift_left(lax.shift_right_logical(i, 6), 4)
            j0 = lax.shift_left(
                lax.bitwise_and(lax.shift_right_logical(i, 4), 3), 4)
            o = lax.bitwise_and(i, 15)
            ccv = cc0 + iota
            jv = j0 + lax.bitwise_and(iota + o, 15)
            wav = jv * 2
            wbv = wav + 1
            a = plsc.load_gather(r0, [ccv, wav])
            b = plsc.load_gather(r0, [ccv, wbv])
            c = plsc.load_gather(r1, [ccv, wav])
            d = plsc.load_gather(r1, [ccv, wbv])
            p = plsc.load_gather(pp, [jv, ccv])
            oa, ob, oc, od, repl = _block_net(a, b, c, d, p)
            plsc.store_scatter(r0, [ccv, wav], oa)
            plsc.store_scatter(r0, [ccv, wbv], ob)
            plsc.store_scatter(r1, [ccv, wav], oc)
            plsc.store_scatter(r1, [ccv, wbv], od)
            plsc.store_scatter(pp, [jv, ccv], repl)

    start_in(jnp.int32(0), 0)
    start_in(jnp.int32(1), 1)

    def outer(u, carry):
        for s in range(3):
            t = u * 3 + s
            wait_in(s)
            compute(s)
            start_out(t, s)
            s2 = (s + 2) % 3

            @pl.when(t >= 1)
            def _():
                wait_out(s2)

            @pl.when(t + 2 < NCHUNK)
            def _():
                start_in(t + 2, s2)
        return carry

    lax.fori_loop(0, NCHUNK // 3, outer, 0)
    wait_out((NCHUNK - 1) % 3)


def kernel(x, pooled):
    xt = x.transpose(0, 1, 3, 2)
    oxt, op2 = _unpool_sc(xt, pooled)
    return oxt.transpose(0, 1, 3, 2), op2
